# baseline (speedup 1.0000x reference)
import functools

import jax
import jax.numpy as jnp
from jax import lax
from jax.experimental import pallas as pl
from jax.experimental.pallas import tpu as pltpu, tpu_sc as plsc

VOCAB = 1000000
EMBED_DIM = 64
BATCH = 4096
HIST = 200

NC = 2
NS = 16
NW = NC * NS

TOTAL = BATCH * HIST
B_PER_W = TOTAL // NW
IDX_MINOR = 128
IDX_ROWS = B_PER_W // IDX_MINOR
GATHERS_PER_CHUNK = 4
CHUNK = GATHERS_PER_CHUNK * IDX_MINOR
NCHUNKS = B_PER_W // CHUNK


def _emb_body(idx_hbm, table_hbm, out_hbm, idx_v, rows_v, gsem):
  wid = lax.axis_index("s") * NC + lax.axis_index("c")
  base = wid * B_PER_W
  pltpu.sync_copy(idx_hbm.at[wid], idx_v)

  def chunk_body(c, carry):
    cps = []
    for q in range(GATHERS_PER_CHUNK):
      cps.append(
          pltpu.async_copy(
              table_hbm.at[idx_v.at[c * GATHERS_PER_CHUNK + q]],
              rows_v.at[pl.ds(q * IDX_MINOR, IDX_MINOR)],
              gsem,
          )
      )
    for cp in cps:
      cp.wait()
    pltpu.sync_copy(rows_v, out_hbm.at[pl.ds(base + c * CHUNK, CHUNK)])
    return carry

  lax.fori_loop(0, NCHUNKS, chunk_body, 0)


@jax.jit
def _emb_lookup(idx, table):
  mesh = plsc.VectorSubcoreMesh(
      core_axis_name="c", subcore_axis_name="s", num_cores=NC, num_subcores=NS
  )
  f = pl.kernel(
      _emb_body,
      out_type=jax.ShapeDtypeStruct((TOTAL, EMBED_DIM), jnp.float32),
      mesh=mesh,
      scratch_types=[
          pltpu.VMEM((IDX_ROWS, IDX_MINOR), jnp.int32),
          pltpu.VMEM((CHUNK, EMBED_DIM), jnp.float32),
          pltpu.SemaphoreType.DMA,
      ],
      compiler_params=pltpu.CompilerParams(use_tc_tiling_on_sc=False),
  )
  return f(idx, table)


def kernel(input, embed_vecs):
  idx = input.reshape(NW, IDX_ROWS, IDX_MINOR).astype(jnp.int32)
  out = _emb_lookup(idx, embed_vecs)
  return out.reshape(BATCH, HIST, EMBED_DIM)


# speedup vs baseline: 1.0193x; 1.0193x over previous
import functools

import jax
import jax.numpy as jnp
from jax import lax
from jax.experimental import pallas as pl
from jax.experimental.pallas import tpu as pltpu, tpu_sc as plsc

VOCAB = 1000000
EMBED_DIM = 64
BATCH = 4096
HIST = 200

NC = 2
NS = 16
NW = NC * NS

TOTAL = BATCH * HIST
B_PER_W = TOTAL // NW
IDX_MINOR = 128
IDX_ROWS = B_PER_W // IDX_MINOR
GATHERS_PER_CHUNK = 4
CHUNK = GATHERS_PER_CHUNK * IDX_MINOR
NCHUNKS = B_PER_W // CHUNK


def _emb_body(idx_hbm, table_hbm, out_hbm, idx_v, rows0, rows1, gsem0, gsem1,
              wsem0, wsem1):
  wid = lax.axis_index("s") * NC + lax.axis_index("c")
  base = wid * B_PER_W
  pltpu.sync_copy(idx_hbm.at[wid], idx_v)

  def fire_gathers(c, buf, sem):
    for q in range(GATHERS_PER_CHUNK):
      pltpu.async_copy(
          table_hbm.at[idx_v.at[c * GATHERS_PER_CHUNK + q]],
          buf.at[pl.ds(q * IDX_MINOR, IDX_MINOR)],
          sem,
      )

  def drain_gathers(buf, sem):
    pltpu.make_async_copy(out_hbm.at[pl.ds(0, CHUNK)], buf, sem).wait()

  def fire_write(c, buf, sem):
    pltpu.async_copy(buf, out_hbm.at[pl.ds(base + c * CHUNK, CHUNK)], sem)

  def drain_write(buf, sem):
    pltpu.make_async_copy(buf, out_hbm.at[pl.ds(0, CHUNK)], sem).wait()

  fire_gathers(0, rows0, gsem0)

  def pair_body(i, carry):
    c0 = 2 * i
    drain_gathers(rows0, gsem0)
    fire_write(c0, rows0, wsem0)

    @pl.when(i > 0)
    def _():
      drain_write(rows1, wsem1)

    fire_gathers(c0 + 1, rows1, gsem1)
    drain_gathers(rows1, gsem1)
    fire_write(c0 + 1, rows1, wsem1)

    @pl.when(i < NCHUNKS // 2 - 1)
    def _():
      drain_write(rows0, wsem0)
      fire_gathers(c0 + 2, rows0, gsem0)

    return carry

  lax.fori_loop(0, NCHUNKS // 2, pair_body, 0)
  drain_write(rows0, wsem0)
  drain_write(rows1, wsem1)


@jax.jit
def _emb_lookup(idx, table):
  mesh = plsc.VectorSubcoreMesh(
      core_axis_name="c", subcore_axis_name="s", num_cores=NC, num_subcores=NS
  )
  f = pl.kernel(
      _emb_body,
      out_type=jax.ShapeDtypeStruct((TOTAL, EMBED_DIM), jnp.float32),
      mesh=mesh,
      scratch_types=[
          pltpu.VMEM((IDX_ROWS, IDX_MINOR), jnp.int32),
          pltpu.VMEM((CHUNK, EMBED_DIM), jnp.float32),
          pltpu.VMEM((CHUNK, EMBED_DIM), jnp.float32),
          pltpu.SemaphoreType.DMA,
          pltpu.SemaphoreType.DMA,
          pltpu.SemaphoreType.DMA,
          pltpu.SemaphoreType.DMA,
      ],
      compiler_params=pltpu.CompilerParams(use_tc_tiling_on_sc=False),
  )
  return f(idx, table)


def kernel(input, embed_vecs):
  idx = input.reshape(NW, IDX_ROWS, IDX_MINOR).astype(jnp.int32)
  out = _emb_lookup(idx, embed_vecs)
  return out.reshape(BATCH, HIST, EMBED_DIM)


# speedup vs baseline: 1.2388x; 1.2153x over previous
import jax
import jax.numpy as jnp
from jax import lax
from jax.experimental import pallas as pl
from jax.experimental.pallas import tpu as pltpu, tpu_sc as plsc

VOCAB = 1000000
EMBED_DIM = 64
BATCH = 4096
HIST = 200

NC = 2
NS = 16
NW = NC * NS

TH = HIST // 8
TB = BATCH // 128
HL = 8
BL = 128
TE = EMBED_DIM // 8
R = 8
L = 128


def _emb_body(idx_hbm, table_hbm, out_hbm, idx_v, rows0, rows1, tb0, tb1,
              gsem0, gsem1, wsem0, wsem1):
  w = lax.axis_index("s") * NC + lax.axis_index("c")

  for th in range(TH):
    pltpu.async_copy(idx_hbm.at[th, w], idx_v.at[th], gsem0)
  for th in range(TH):
    pltpu.make_async_copy(idx_hbm.at[th, w], idx_v.at[th], gsem0).wait()

  def fire_gather(h, rows, sem):
    pltpu.async_copy(table_hbm.at[idx_v.at[h // HL, h % HL]], rows, sem)

  def drain_gather(rows, sem):
    pltpu.make_async_copy(table_hbm.at[idx_v.at[0, 0]], rows, sem).wait()

  lanes = lax.iota(jnp.int32, 16)

  def transpose_unit(rows, tbuf):
    def kstep(k, carry):
      perm = lax.bitwise_and(lanes + k, 15)
      for b0 in range(0, BL, 16):
        for e0 in range(0, EMBED_DIM, 16):
          vals = plsc.load_gather(rows, [b0 + lanes, e0 + perm])
          plsc.store_scatter(tbuf, [e0 + perm, b0 + lanes], vals)
      return carry
    lax.fori_loop(0, 16, kstep, 0)

  def fire_writes(h, tbuf, sem):
    for te in range(TE):
      pltpu.async_copy(tbuf.at[pl.ds(te * R, R)], out_hbm.at[h, te, w], sem)

  def drain_writes(h, tbuf, sem):
    for te in range(TE):
      pltpu.make_async_copy(
          tbuf.at[pl.ds(te * R, R)], out_hbm.at[h, te, w], sem).wait()

  fire_gather(0, rows0, gsem0)

  def pair_body(i, carry):
    h0 = 2 * i
    drain_gather(rows0, gsem0)

    @pl.when(i > 0)
    def _():
      drain_writes(h0, tb0, wsem0)

    fire_gather(h0 + 1, rows1, gsem1)
    transpose_unit(rows0, tb0)
    fire_writes(h0, tb0, wsem0)
    drain_gather(rows1, gsem1)

    @pl.when(i < HIST // 2 - 1)
    def _():
      fire_gather(h0 + 2, rows0, gsem0)

    @pl.when(i > 0)
    def _():
      drain_writes(h0, tb1, wsem1)

    transpose_unit(rows1, tb1)
    fire_writes(h0 + 1, tb1, wsem1)
    return carry

  lax.fori_loop(0, HIST // 2, pair_body, 0)
  drain_writes(0, tb0, wsem0)
  drain_writes(0, tb1, wsem1)


@jax.jit
def _emb_lookup(idx, table):
  mesh = plsc.VectorSubcoreMesh(
      core_axis_name="c", subcore_axis_name="s", num_cores=NC, num_subcores=NS
  )
  f = pl.kernel(
      _emb_body,
      out_type=jax.ShapeDtypeStruct((HIST, TE, TB, R, L), jnp.float32),
      mesh=mesh,
      scratch_types=[
          pltpu.VMEM((TH, HL, BL), jnp.int32),
          pltpu.VMEM((BL, EMBED_DIM), jnp.float32),
          pltpu.VMEM((BL, EMBED_DIM), jnp.float32),
          pltpu.VMEM((EMBED_DIM, BL), jnp.float32),
          pltpu.VMEM((EMBED_DIM, BL), jnp.float32),
          pltpu.SemaphoreType.DMA,
          pltpu.SemaphoreType.DMA,
          pltpu.SemaphoreType.DMA,
          pltpu.SemaphoreType.DMA,
      ],
      compiler_params=pltpu.CompilerParams(
          use_tc_tiling_on_sc=False, needs_layout_passes=False),
  )
  return f(idx, table)


def kernel(input, embed_vecs):
  idx = input.astype(jnp.int32).reshape(TB, BL, TH, HL).transpose(2, 0, 3, 1)
  out5 = _emb_lookup(idx, embed_vecs)
  return out5.transpose(2, 4, 0, 1, 3).reshape(BATCH, HIST, EMBED_DIM)
